# baseline (speedup 1.0000x reference)
import jax
import jax.numpy as jnp
from jax import lax
from jax.experimental import pallas as pl
from jax.experimental.pallas import tpu as pltpu

_CB = 16
_NBUF = 4


def _body(dpt_ref, hpt_ref, wpt_ref, out_hbm, vbuf, sems, sems2):
    ncs = pl.num_programs(0)
    pc = pl.program_id(0)
    slot = lax.rem(pc, _NBUF)
    cb, d = dpt_ref.shape
    hw = out_hbm.shape[3]

    @pl.when(pc >= _NBUF)
    def _():
        pltpu.make_async_copy(
            vbuf.at[slot], out_hbm.at[0, pl.ds(0, _CB)], sems.at[slot]
        ).wait()

    @pl.when(pc >= 2 * _NBUF)
    def _():
        pltpu.make_async_copy(
            vbuf.at[slot], out_hbm.at[1, pl.ds(0, _CB)], sems2.at[slot]
        ).wait()

    @pl.when(pc >= _NBUF)
    def _():
        prev = (pc - _NBUF) * _CB
        pltpu.make_async_copy(
            out_hbm.at[0, pl.ds(prev, _CB)],
            out_hbm.at[1, pl.ds(prev, _CB)],
            sems2.at[slot],
        ).start()

    h = hpt_ref[pl.ds(pc * _CB, _CB), :]
    w = wpt_ref[pl.ds(pc * _CB, _CB), :]
    hwsum = (h[:, :, None] + w[:, None, :]).reshape(_CB, hw)
    dv = dpt_ref[pl.ds(pc * _CB, _CB), :]
    for i in range(d):
        vbuf[slot, :, i, :] = hwsum + dv[:, i][:, None]

    pltpu.make_async_copy(
        vbuf.at[slot], out_hbm.at[0, pl.ds(pc * _CB, _CB)], sems.at[slot]
    ).start()

    @pl.when(pc == ncs - 1)
    def _():
        for s in range(_NBUF):
            step = ncs - _NBUF + s
            pltpu.make_async_copy(
                vbuf.at[s], out_hbm.at[0, pl.ds(0, _CB)], sems.at[s]
            ).wait()
            pltpu.make_async_copy(
                out_hbm.at[0, pl.ds(step * _CB, _CB)],
                out_hbm.at[1, pl.ds(step * _CB, _CB)],
                sems2.at[s],
            ).start()
        for s in range(_NBUF):
            for _u in range(2):
                pltpu.make_async_copy(
                    vbuf.at[s], out_hbm.at[1, pl.ds(0, _CB)], sems2.at[s]
                ).wait()


def kernel(x, d_weight, h_weight, w_weight):
    B = x.shape[0]
    d, h, w = x.shape[-3:]
    c = d_weight.shape[1]
    C = 256

    f32 = jnp.float32
    dpt = jnp.zeros((C, d), f32).at[0:c, :].set(d_weight[:d].T.astype(f32))
    hpt = jnp.zeros((C, h), f32).at[c:2 * c, :].set(h_weight[:h].T.astype(f32))
    wpt = jnp.zeros((C, w), f32).at[2 * c:C, :].set(
        w_weight[:w, : C - 2 * c].T.astype(f32))

    grid = (C // _CB,)
    out4 = pl.pallas_call(
        _body,
        grid=grid,
        in_specs=[
            pl.BlockSpec((C, d), lambda pc: (0, 0)),
            pl.BlockSpec((C, h), lambda pc: (0, 0)),
            pl.BlockSpec((C, w), lambda pc: (0, 0)),
        ],
        out_specs=pl.BlockSpec(memory_space=pltpu.HBM),
        out_shape=jax.ShapeDtypeStruct((B, C, d, h * w), f32),
        scratch_shapes=[
            pltpu.VMEM((_NBUF, _CB, d, h * w), f32),
            pltpu.SemaphoreType.DMA((_NBUF,)),
            pltpu.SemaphoreType.DMA((_NBUF,)),
        ],
    )(dpt, hpt, wpt)
    return out4.reshape(B, C, d, h, w)


# speedup vs baseline: 11.8061x; 11.8061x over previous
import jax
import jax.numpy as jnp
from jax import lax
from jax.experimental import pallas as pl
from jax.experimental.pallas import tpu as pltpu

_CB = 16
_NBUF = 4


def _body(dpt_ref, hpt_ref, wpt_ref, out_hbm, vbuf, sems):
    ncs = pl.num_programs(0)
    pc = pl.program_id(0)
    slot = lax.rem(pc, _NBUF)
    nb = out_hbm.shape[0]
    cb, d = dpt_ref.shape
    hw = out_hbm.shape[3]

    @pl.when(pc >= _NBUF)
    def _():
        pltpu.make_async_copy(
            vbuf.at[slot], out_hbm.at[:, pl.ds(0, _CB)], sems.at[slot]
        ).wait()

    h = hpt_ref[pl.ds(pc * _CB, _CB), :]
    w = wpt_ref[pl.ds(pc * _CB, _CB), :]
    hwsum = (h[:, :, None] + w[:, None, :]).reshape(_CB, hw)
    dv = dpt_ref[pl.ds(pc * _CB, _CB), :]
    for i in range(d):
        row = hwsum + dv[:, i][:, None]
        for b in range(nb):
            vbuf[slot, b, :, i, :] = row

    for b in range(nb):
        pltpu.make_async_copy(
            vbuf.at[slot, b], out_hbm.at[b, pl.ds(pc * _CB, _CB)],
            sems.at[slot]
        ).start()

    @pl.when(pc == ncs - 1)
    def _():
        for s in range(_NBUF):
            pltpu.make_async_copy(
                vbuf.at[s], out_hbm.at[:, pl.ds(0, _CB)], sems.at[s]
            ).wait()


def kernel(x, d_weight, h_weight, w_weight):
    B = x.shape[0]
    d, h, w = x.shape[-3:]
    c = d_weight.shape[1]
    C = 256

    f32 = jnp.float32
    dpt = jnp.zeros((C, d), f32).at[0:c, :].set(d_weight[:d].T.astype(f32))
    hpt = jnp.zeros((C, h), f32).at[c:2 * c, :].set(h_weight[:h].T.astype(f32))
    wpt = jnp.zeros((C, w), f32).at[2 * c:C, :].set(
        w_weight[:w, : C - 2 * c].T.astype(f32))

    grid = (C // _CB,)
    out4 = pl.pallas_call(
        _body,
        grid=grid,
        in_specs=[
            pl.BlockSpec((C, d), lambda pc: (0, 0)),
            pl.BlockSpec((C, h), lambda pc: (0, 0)),
            pl.BlockSpec((C, w), lambda pc: (0, 0)),
        ],
        out_specs=pl.BlockSpec(memory_space=pltpu.HBM),
        out_shape=jax.ShapeDtypeStruct((B, C, d, h * w), f32),
        scratch_shapes=[
            pltpu.VMEM((_NBUF, B, _CB, d, h * w), f32),
            pltpu.SemaphoreType.DMA((_NBUF,)),
        ],
    )(dpt, hpt, wpt)
    return out4.reshape(B, C, d, h, w)
